# baseline (speedup 1.0000x reference)
import functools

import jax
import jax.numpy as jnp
from jax import lax
from jax.experimental import pallas as pl
from jax.experimental.pallas import tpu as pltpu
from jax.experimental.pallas import tpu_sc as plsc

NC, NS = 2, 16
NW = NC * NS
CHUNK = 128
DP = 304


@functools.partial(jax.jit, static_argnames=("n_chunks", "dim"))
def _sc_gather(idx, table_pad, n_chunks, dim):
    @functools.partial(
        pl.kernel,
        out_type=jax.ShapeDtypeStruct((NW * n_chunks * CHUNK, DP), jnp.float32),
        mesh=plsc.VectorSubcoreMesh(core_axis_name="c", subcore_axis_name="s"),
        compiler_params=pltpu.CompilerParams(use_tc_tiling_on_sc=False),
        scratch_types=[
            pltpu.VMEM((n_chunks, CHUNK), jnp.int32),
            pltpu.VMEM((CHUNK, DP), jnp.float32),
            pltpu.VMEM((CHUNK, DP), jnp.float32),
            pltpu.SemaphoreType.DMA,
            pltpu.SemaphoreType.DMA,
        ],
    )
    def k(idx_hbm, table_hbm, out_hbm, idx_v, buf0, buf1, g0, g1):
        wid = lax.axis_index("s") * NC + lax.axis_index("c")
        base = wid * n_chunks * CHUNK
        pltpu.sync_copy(idx_hbm.at[wid], idx_v)

        def gather(j, buf, sem):
            return pltpu.make_async_copy(table_hbm.at[idx_v.at[j]], buf, sem)

        def scatter(j, buf):
            pltpu.sync_copy(buf, out_hbm.at[pl.ds(base + j * CHUNK, CHUNK)])

        gather(0, buf0, g0).start()

        def body(t, carry):
            j = 2 * t
            gather(j + 1, buf1, g1).start()
            gather(j, buf0, g0).wait()
            scatter(j, buf0)

            @pl.when(t < n_chunks // 2 - 1)
            def _():
                gather(j + 2, buf0, g0).start()

            gather(j + 1, buf1, g1).wait()
            scatter(j + 1, buf1)
            return carry

        lax.fori_loop(0, n_chunks // 2, body, 0)

    return k(idx, table_pad)


def kernel(indices, table):
    batch, seq = indices.shape
    vocab, dim = table.shape
    total = batch * seq
    assert total % (NW * CHUNK) == 0 and dim <= DP
    n_chunks = total // (NW * CHUNK)
    idx = indices.reshape(NW, n_chunks, CHUNK)
    table_pad = jnp.pad(table, ((0, 0), (0, DP - dim)))
    out = _sc_gather(idx, table_pad, n_chunks, dim)
    return out[:, :dim].reshape(batch, seq, dim)
